# baseline (speedup 1.0000x reference)
import functools

import jax
import jax.numpy as jnp
from jax import lax
from jax.experimental import pallas as pl
from jax.experimental.pallas import tpu as pltpu
from jax.experimental.pallas import tpu_sc as plsc

D_MODEL = 64
NC = 2
NS = 16
NW = NC * NS
SB = 4096 // 128
NST = 200 // 8
NCH = 50
CHR = 512


def _embed(xt3, table):
    mesh = plsc.VectorSubcoreMesh(
        core_axis_name="c", subcore_axis_name="s", num_cores=NC, num_subcores=NS
    )

    @functools.partial(
        pl.kernel,
        out_type=jax.ShapeDtypeStruct((200, 8, 32, 8, 128), jnp.float32),
        mesh=mesh,
        scratch_types=[
            pltpu.VMEM((CHR,), jnp.int32),
            pltpu.VMEM((CHR,), jnp.int32),
            pltpu.VMEM((CHR, D_MODEL), jnp.float32),
            pltpu.VMEM((CHR, D_MODEL), jnp.float32),
            pltpu.VMEM((4, 8, 8, 128), jnp.float32),
            pltpu.SemaphoreType.DMA,
            pltpu.SemaphoreType.DMA,
            pltpu.SemaphoreType.DMA,
            pltpu.SemaphoreType.DMA,
            pltpu.SemaphoreType.DMA,
        ],
        compiler_params=pltpu.CompilerParams(use_tc_tiling_on_sc=False, needs_layout_passes=False),
    )
    def k(x_hbm, tab_hbm, out_hbm, ix0, ix1, r0, r1, outb,
          is0, is1, gs0, gs1, wsem):
        idx_v = (ix0, ix1)
        rows_v = (r0, r1)
        isem = (is0, is1)
        gsem = (gs0, gs1)
        wid = lax.axis_index("s") * NC + lax.axis_index("c")
        iot = lax.iota(jnp.int32, 16)
        zc = jnp.zeros((16,), jnp.int32)

        def start_idx(c, b):
            st = c // 2
            h = c % 2
            pltpu.async_copy(
                x_hbm.at[st, wid, pl.ds(h * CHR, CHR)], idx_v[b], isem[b])

        def wait_idx(b):
            pltpu.make_async_copy(
                x_hbm.at[0, 0, pl.ds(0, CHR)], idx_v[b], isem[b]).wait()

        def start_gather(b):
            pltpu.async_copy(tab_hbm.at[idx_v[b]], rows_v[b], gsem[b])

        def wait_gather(b):
            pltpu.make_async_copy(
                tab_hbm.at[idx_v[b]], rows_v[b], gsem[b]).wait()

        def start_write(c):
            pltpu.async_copy(
                outb, out_hbm.at[pl.ds(c * 4, 4), slice(None), wid], wsem)

        def wait_write():
            pltpu.make_async_copy(
                outb, out_hbm.at[pl.ds(0, 4), slice(None), 0], wsem).wait()

        def transpose(b):
            rows = rows_v[b]

            @pl.loop(0, 256, unroll=8)
            def _(i):
                si4 = i >> 6
                g = (i >> 3) & 7
                dt = i & 7
                row_idx = si4 * 128 + g * 16 + iot
                colb = zc + dt * 8
                for di in range(8):
                    vals = plsc.load_gather(rows, [row_idx, colb + di])
                    outb[si4, dt, di, pl.ds(g * 16, 16)] = vals

        def body(c, b, first, last):
            wait_gather(b)
            if not last:
                start_idx(c + 2, b)
            if not first:
                wait_write()
            plsc.subcore_barrier()
            transpose(b)
            start_write(c)
            if not last:
                wait_idx(b)
                start_gather(b)

        start_idx(0, 0)
        start_idx(1, 1)
        wait_idx(0)
        start_gather(0)
        wait_idx(1)
        start_gather(1)

        body(0, 0, True, False)
        body(1, 1, True, False)

        @pl.loop(2, NCH - 2, step=2)
        def _(t):
            for b in (0, 1):
                body(t + b, b, False, False)

        body(NCH - 2, 0, False, True)
        body(NCH - 1, 1, False, True)
        wait_write()

    return k(xt3, table)


def kernel(x, table):
    xt3 = x.reshape(32, 128, 25, 8).transpose(2, 0, 3, 1).reshape(25, 32, 1024)
    out5 = _embed(xt3, table)
    return out5.transpose(2, 4, 0, 1, 3).reshape(4096, 200, 64)


# speedup vs baseline: 1.6179x; 1.6179x over previous
import functools

import jax
import jax.numpy as jnp
from jax import lax
from jax.experimental import pallas as pl
from jax.experimental.pallas import tpu as pltpu
from jax.experimental.pallas import tpu_sc as plsc

D_MODEL = 64
NC = 2
NS = 16
NW = NC * NS
SB = 4096 // 128
NST = 200 // 8
NCH = 50
CHR = 512


def _embed(xt3, table):
    mesh = plsc.VectorSubcoreMesh(
        core_axis_name="c", subcore_axis_name="s", num_cores=NC, num_subcores=NS
    )

    @functools.partial(
        pl.kernel,
        out_type=jax.ShapeDtypeStruct((200, 8, 32, 8, 128), jnp.float32),
        mesh=mesh,
        scratch_types=[
            pltpu.VMEM((CHR,), jnp.int32),
            pltpu.VMEM((CHR,), jnp.int32),
            pltpu.VMEM((CHR, D_MODEL), jnp.float32),
            pltpu.VMEM((CHR, D_MODEL), jnp.float32),
            pltpu.VMEM((4, 8, 8, 128), jnp.float32),
            pltpu.SemaphoreType.DMA,
            pltpu.SemaphoreType.DMA,
            pltpu.SemaphoreType.DMA,
            pltpu.SemaphoreType.DMA,
            pltpu.SemaphoreType.DMA,
        ],
        compiler_params=pltpu.CompilerParams(use_tc_tiling_on_sc=False, needs_layout_passes=False),
    )
    def k(x_hbm, tab_hbm, out_hbm, ix0, ix1, r0, r1, outb,
          is0, is1, gs0, gs1, wsem):
        idx_v = (ix0, ix1)
        rows_v = (r0, r1)
        isem = (is0, is1)
        gsem = (gs0, gs1)
        wid = lax.axis_index("s") * NC + lax.axis_index("c")
        iot = lax.iota(jnp.int32, 16)
        zc = jnp.zeros((16,), jnp.int32)

        def start_idx(c, b):
            st = c // 2
            h = c % 2
            pltpu.async_copy(
                x_hbm.at[st, wid, pl.ds(h * CHR, CHR)], idx_v[b], isem[b])

        def wait_idx(b):
            pltpu.make_async_copy(
                x_hbm.at[0, 0, pl.ds(0, CHR)], idx_v[b], isem[b]).wait()

        def start_gather(b):
            pltpu.async_copy(tab_hbm.at[idx_v[b]], rows_v[b], gsem[b])

        def wait_gather(b):
            pltpu.make_async_copy(
                tab_hbm.at[idx_v[b]], rows_v[b], gsem[b]).wait()

        def start_write(c):
            pltpu.async_copy(
                outb, out_hbm.at[pl.ds(c * 4, 4), slice(None), wid], wsem)

        def wait_write():
            pltpu.make_async_copy(
                outb, out_hbm.at[pl.ds(0, 4), slice(None), 0], wsem).wait()

        def transpose(b):
            rows = rows_v[b]

            @pl.loop(0, 256, unroll=8)
            def _(i):
                si4 = i >> 6
                g = (i >> 3) & 7
                dt = i & 7
                row_idx = si4 * 128 + g * 16 + iot
                colb = zc + dt * 8
                vals = [plsc.load_gather(rows, [row_idx, colb + di])
                        for di in range(8)]
                for di in range(8):
                    outb[si4, dt, di, pl.ds(g * 16, 16)] = vals[di]

        def body(c, b, first, last):
            wait_gather(b)
            if not last:
                start_idx(c + 2, b)
            if not first:
                wait_write()
            plsc.subcore_barrier()
            transpose(b)
            start_write(c)
            if not last:
                wait_idx(b)
                start_gather(b)

        start_idx(0, 0)
        start_idx(1, 1)
        wait_idx(0)
        start_gather(0)
        wait_idx(1)
        start_gather(1)

        body(0, 0, True, False)
        body(1, 1, True, False)

        @pl.loop(2, NCH - 2, step=2)
        def _(t):
            for b in (0, 1):
                body(t + b, b, False, False)

        body(NCH - 2, 0, False, True)
        body(NCH - 1, 1, False, True)
        wait_write()

    return k(xt3, table)


def kernel(x, table):
    xt3 = x.reshape(32, 128, 25, 8).transpose(2, 0, 3, 1).reshape(25, 32, 1024)
    out5 = _embed(xt3, table)
    return out5.transpose(2, 4, 0, 1, 3).reshape(4096, 200, 64)


# speedup vs baseline: 5.3727x; 3.3209x over previous
import functools

import jax
import jax.numpy as jnp
from jax import lax
from jax.experimental import pallas as pl
from jax.experimental.pallas import tpu as pltpu
from jax.experimental.pallas import tpu_sc as plsc

D_MODEL = 64
NC = 2
NS = 16
NW = NC * NS
SB = 4096 // 128
NST = 200 // 8
NCH = 50
CHR = 512


def _embed(xt3, table):
    mesh = plsc.VectorSubcoreMesh(
        core_axis_name="c", subcore_axis_name="s", num_cores=NC, num_subcores=NS
    )

    @functools.partial(
        pl.kernel,
        out_type=jax.ShapeDtypeStruct((200 * 8 * 32 * 8 * 128,), jnp.float32),
        mesh=mesh,
        scratch_types=[
            pltpu.VMEM((CHR,), jnp.int32),
            pltpu.VMEM((CHR,), jnp.int32),
            pltpu.VMEM((CHR, D_MODEL), jnp.float32),
            pltpu.VMEM((CHR, D_MODEL), jnp.float32),
            pltpu.VMEM((4 * 8 * 8 * 128,), jnp.float32),
            pltpu.SemaphoreType.DMA,
            pltpu.SemaphoreType.DMA,
            pltpu.SemaphoreType.DMA,
            pltpu.SemaphoreType.DMA,
            pltpu.SemaphoreType.DMA,
        ],
        compiler_params=pltpu.CompilerParams(use_tc_tiling_on_sc=False, needs_layout_passes=False),
    )
    def k(x_hbm, tab_hbm, out_hbm, ix0, ix1, r0, r1, outb,
          is0, is1, gs0, gs1, wsem):
        idx_v = (ix0, ix1)
        rows_v = (r0, r1)
        isem = (is0, is1)
        gsem = (gs0, gs1)
        wid = lax.axis_index("s") * NC + lax.axis_index("c")
        iot = lax.iota(jnp.int32, 16)
        zc = jnp.zeros((16,), jnp.int32)

        def start_idx(c, b):
            st = c // 2
            h = c % 2
            pltpu.async_copy(
                x_hbm.at[st, wid, pl.ds(h * CHR, CHR)], idx_v[b], isem[b])

        def wait_idx(b):
            pltpu.make_async_copy(
                x_hbm.at[0, 0, pl.ds(0, CHR)], idx_v[b], isem[b]).wait()

        def start_gather(b):
            pltpu.async_copy(tab_hbm.at[idx_v[b]], rows_v[b], gsem[b])

        def wait_gather(b):
            pltpu.make_async_copy(
                tab_hbm.at[idx_v[b]], rows_v[b], gsem[b]).wait()

        def start_write(c):
            for j in range(32):
                si4, dt = j >> 3, j & 7
                off = (((c * 4 + si4) * 8 + dt) * 32 + wid) * 1024
                pltpu.async_copy(
                    outb.at[pl.ds(j * 1024, 1024)],
                    out_hbm.at[pl.ds(off, 1024)], wsem)

        def wait_write():
            for j in range(32):
                pltpu.make_async_copy(
                    outb.at[pl.ds(0, 1024)],
                    out_hbm.at[pl.ds(0, 1024)], wsem).wait()

        pks = [(iot + k) & 15 for k in range(16)]
        qks = [((pk >> 3) << 10) + ((pk & 7) << 7) + iot for pk in pks]

        def transpose(b):
            rows = rows_v[b]

            @pl.loop(0, 128, unroll=4)
            def _(i):
                si4 = i >> 5
                g = (i >> 2) & 7
                dblk = i & 3
                row_idx = si4 * 128 + g * 16 + iot
                dest_base = si4 * 8192 + dblk * 2048 + g * 16
                vals = [plsc.load_gather(rows, [row_idx, pks[k] + dblk * 16])
                        for k in range(16)]
                for k in range(16):
                    plsc.store_scatter(outb, [qks[k] + dest_base], vals[k])

        def body(c, b, first, last):
            wait_gather(b)
            if not last:
                start_idx(c + 2, b)
            if not first:
                wait_write()
            plsc.subcore_barrier()
            transpose(b)
            start_write(c)
            if not last:
                wait_idx(b)
                start_gather(b)

        start_idx(0, 0)
        start_idx(1, 1)
        wait_idx(0)
        start_gather(0)
        wait_idx(1)
        start_gather(1)

        body(0, 0, True, False)
        body(1, 1, True, False)

        @pl.loop(2, NCH - 2, step=2)
        def _(t):
            for b in (0, 1):
                body(t + b, b, False, False)

        body(NCH - 2, 0, False, True)
        body(NCH - 1, 1, False, True)
        wait_write()

    return k(xt3, table)


def kernel(x, table):
    xt3 = x.reshape(32, 128, 25, 8).transpose(2, 0, 3, 1).reshape(25, 32, 1024)
    out5 = _embed(xt3, table).reshape(200, 8, 32, 8, 128)
    return out5.transpose(2, 4, 0, 1, 3).reshape(4096, 200, 64)
